# baseline (speedup 1.0000x reference)
import functools

import jax
import jax.numpy as jnp
from jax import lax
from jax.experimental import pallas as pl
from jax.experimental.pallas import tpu as pltpu, tpu_sc as plsc

_FMIN = float(jnp.finfo(jnp.float32).min)

_NC, _NS, _L = 2, 16, 16
_NW = _NC * _NS


def _sc_routing_body(preds_hbm, scores_hbm, argmax_hbm, p_v, s_v, a_v,
                     *, B, K, N, V):
    WPB = _NW // B
    Nw = N // WPB
    cid = lax.axis_index("c")
    sid = lax.axis_index("s")
    wid = cid * _NS + sid
    b = wid // WPB
    v0 = (wid % WPB) * Nw

    def chunk_body(ci, _):
        base = v0 + ci * V
        pltpu.sync_copy(preds_hbm.at[b, :, pl.ds(base, V)], p_v)

        def group_body(g, carry):
            sl = pl.ds(g * _L, _L)
            m = p_v[0, sl]
            am = jnp.zeros((_L,), jnp.int32)
            for k in range(1, K):
                pk = p_v[k, sl]
                gt = pk > m
                m = jnp.where(gt, pk, m)
                am = jnp.where(gt, jnp.full((_L,), k, jnp.int32), am)
            s_v[sl] = m
            a_v[sl] = am
            return carry

        lax.fori_loop(0, V // _L, group_body, 0, unroll=2)
        pltpu.sync_copy(s_v, scores_hbm.at[b, pl.ds(base, V)])
        pltpu.sync_copy(a_v, argmax_hbm.at[b, pl.ds(base, V)])
        return 0

    lax.fori_loop(0, Nw // V, chunk_body, 0, unroll=False)


def _sc_routing(pf, V=2048):
    B, K, N = pf.shape
    mesh = plsc.VectorSubcoreMesh(core_axis_name="c", subcore_axis_name="s")
    fn = pl.kernel(
        functools.partial(_sc_routing_body, B=B, K=K, N=N, V=V),
        mesh=mesh,
        out_type=[
            jax.ShapeDtypeStruct((B, N), jnp.float32),
            jax.ShapeDtypeStruct((B, N), jnp.int32),
        ],
        scratch_types=[
            pltpu.VMEM((K, V), jnp.float32),
            pltpu.VMEM((V,), jnp.float32),
            pltpu.VMEM((V,), jnp.int32),
        ],
    )
    return fn(pf)


def _stats_apply_body(scores_ref, am_ref, x_ref, out_ref, segmax_s,
                      denom_s, fsum_s, *, K):
    ph = pl.program_id(1)
    nb = pl.program_id(2)

    @pl.when(ph == 0)
    def _phase0():
        @pl.when(nb == 0)
        def _():
            segmax_s[...] = jnp.full(segmax_s.shape, _FMIN, jnp.float32)
            denom_s[...] = jnp.zeros(denom_s.shape, jnp.float32)
            fsum_s[...] = jnp.zeros(fsum_s.shape, jnp.float32)

        s = scores_ref[0]
        am = am_ref[0]
        xb = x_ref[0]
        kio = jax.lax.broadcasted_iota(jnp.int32, (K,) + am.shape[1:], 0)
        ohf = (kio == am).astype(jnp.float32)
        lm = jnp.max(jnp.where(ohf > 0, s, _FMIN), axis=1, keepdims=True)
        old = segmax_s[...]
        new = jnp.maximum(old, lm)
        segmax_s[...] = new
        scale = jnp.exp(old - new)
        smg = jax.lax.dot_general(
            new.T, ohf, (((1,), (0,)), ((), ())),
            preferred_element_type=jnp.float32)
        w = jnp.exp(s - smg)
        wog = ohf * w
        denom_s[...] = denom_s[...] * scale + jnp.sum(wog, axis=1,
                                                      keepdims=True)
        fsum_s[...] = fsum_s[...] * scale + jax.lax.dot_general(
            wog, xb, (((1,), (1,)), ((), ())),
            preferred_element_type=jnp.float32)

    @pl.when(ph == 1)
    def _phase1():
        xb = x_ref[0]
        am = am_ref[0]
        denom = denom_s[...]
        cls = fsum_s[...] / jnp.where(denom > 0, denom, 1.0)
        kio = jax.lax.broadcasted_iota(jnp.int32, (K,) + am.shape[1:], 0)
        ohf = (kio == am).astype(jnp.float32)
        sl = jax.lax.dot_general(
            cls, ohf, (((0,), (0,)), ((), ())),
            preferred_element_type=jnp.float32)
        out_ref[0] = xb * sl


def _tc_stats_apply(xf, scores, am, Nb=8192):
    B, C, N = xf.shape
    K = 16
    NB = N // Nb
    sc3 = scores.reshape(B, 1, N)
    am3 = am.reshape(B, 1, N)
    return pl.pallas_call(
        functools.partial(_stats_apply_body, K=K),
        grid=(B, 2, NB),
        in_specs=[
            pl.BlockSpec((1, 1, Nb), lambda b, ph, n: (b, 0, n)),
            pl.BlockSpec((1, 1, Nb), lambda b, ph, n: (b, 0, n)),
            pl.BlockSpec((1, C, Nb), lambda b, ph, n: (b, 0, n)),
        ],
        out_specs=pl.BlockSpec(
            (1, C, Nb), lambda b, ph, n: (b, 0, jnp.where(ph == 1, n, 0))),
        out_shape=jax.ShapeDtypeStruct((B, C, N), jnp.float32),
        scratch_shapes=[
            pltpu.VMEM((K, 1), jnp.float32),
            pltpu.VMEM((K, 1), jnp.float32),
            pltpu.VMEM((K, C), jnp.float32),
        ],
    )(sc3, am3, xf)


def kernel(x, preds):
    B, C, H, W, D = x.shape
    N = H * W * D
    K = preds.shape[1]
    pf = preds.reshape(B, K, N)
    xf = x.reshape(B, C, N)
    scores, am = _sc_routing(pf)
    out = _tc_stats_apply(xf, scores, am)
    return out.reshape(B, C, H, W, D)


# speedup vs baseline: 1.1117x; 1.1117x over previous
import functools

import jax
import jax.numpy as jnp
from jax.experimental import pallas as pl
from jax.experimental.pallas import tpu as pltpu

_FMIN = float(jnp.finfo(jnp.float32).min)


def _fused_body(preds_ref, x_ref, out_ref, xstash, am_s, segmax_s,
                denom_s, fsum_s, *, K):
    ph = pl.program_id(1)
    nb = pl.program_id(2)

    @pl.when(ph == 0)
    def _phase0():
        @pl.when(nb == 0)
        def _():
            segmax_s[...] = jnp.full(segmax_s.shape, _FMIN, jnp.float32)
            denom_s[...] = jnp.zeros(denom_s.shape, jnp.float32)
            fsum_s[...] = jnp.zeros(fsum_s.shape, jnp.float32)

        p = preds_ref[0]
        xb = x_ref[0]
        xstash[nb] = xb
        m = jnp.max(p, axis=0, keepdims=True)
        kio = jax.lax.broadcasted_iota(jnp.int32, p.shape, 0)
        am = jnp.min(jnp.where(p == m, kio, K), axis=0, keepdims=True)
        am_s[nb] = am
        ohf = (kio == am).astype(jnp.float32)
        lm = jnp.max(jnp.where(ohf > 0, m, _FMIN), axis=1, keepdims=True)
        old = segmax_s[...]
        new = jnp.maximum(old, lm)
        segmax_s[...] = new
        scale = jnp.exp(old - new)
        smg = jax.lax.dot_general(
            new.T, ohf, (((1,), (0,)), ((), ())),
            preferred_element_type=jnp.float32)
        w = jnp.exp(m - smg)
        wog = ohf * w
        denom_s[...] = denom_s[...] * scale + jnp.sum(wog, axis=1,
                                                      keepdims=True)
        fsum_s[...] = fsum_s[...] * scale + jax.lax.dot_general(
            wog, xb, (((1,), (1,)), ((), ())),
            preferred_element_type=jnp.float32)

    @pl.when(ph == 1)
    def _phase1():
        xb = xstash[nb]
        am = am_s[nb]
        denom = denom_s[...]
        cls = fsum_s[...] / jnp.where(denom > 0, denom, 1.0)
        kio = jax.lax.broadcasted_iota(jnp.int32, (K,) + am.shape[1:], 0)
        ohf = (kio == am).astype(jnp.float32)
        sl = jax.lax.dot_general(
            cls, ohf, (((0,), (0,)), ((), ())),
            preferred_element_type=jnp.float32)
        out_ref[0] = xb * sl


def kernel(x, preds):
    B, C, H, W, D = x.shape
    K = preds.shape[1]
    N = H * W * D
    Nb = 8192
    NB = N // Nb
    xf = x.reshape(B, C, N)
    pf = preds.reshape(B, K, N)

    out = pl.pallas_call(
        functools.partial(_fused_body, K=K),
        grid=(B, 2, NB),
        in_specs=[
            pl.BlockSpec((1, K, Nb),
                         lambda b, ph, n: (b, 0, jnp.where(ph == 0, n, 0))),
            pl.BlockSpec((1, C, Nb),
                         lambda b, ph, n: (b, 0, jnp.where(ph == 0, n, 0))),
        ],
        out_specs=pl.BlockSpec(
            (1, C, Nb), lambda b, ph, n: (b, 0, jnp.where(ph == 1, n, 0))),
        out_shape=jax.ShapeDtypeStruct((B, C, N), jnp.float32),
        scratch_shapes=[
            pltpu.VMEM((NB, C, Nb), jnp.float32),
            pltpu.VMEM((NB, 1, Nb), jnp.int32),
            pltpu.VMEM((K, 1), jnp.float32),
            pltpu.VMEM((K, 1), jnp.float32),
            pltpu.VMEM((K, C), jnp.float32),
        ],
    )(pf, xf)

    return out.reshape(B, C, H, W, D)


# speedup vs baseline: 1.2127x; 1.0909x over previous
import functools

import jax
import jax.numpy as jnp
from jax.experimental import pallas as pl
from jax.experimental.pallas import tpu as pltpu

_FMIN = float(jnp.finfo(jnp.float32).min)


def _fused_body(preds_ref, x_ref, out_ref, xstash, am_s, segmax_s,
                denom_s, fsum_s, *, K):
    ph = pl.program_id(1)
    nb = pl.program_id(2)

    @pl.when(ph == 0)
    def _phase0():
        @pl.when(nb == 0)
        def _():
            segmax_s[...] = jnp.full(segmax_s.shape, _FMIN, jnp.float32)
            denom_s[...] = jnp.zeros(denom_s.shape, jnp.float32)
            fsum_s[...] = jnp.zeros(fsum_s.shape, jnp.float32)

        p = preds_ref[0]
        xb = x_ref[0]
        xstash[nb] = xb
        m = jnp.max(p, axis=0, keepdims=True)
        kio = jax.lax.broadcasted_iota(jnp.int32, p.shape, 0)
        am = jnp.min(jnp.where(p == m, kio, K), axis=0, keepdims=True)
        am_s[nb] = am.reshape(am_s.shape[1:])
        ohf = (kio == am).astype(jnp.float32)
        lm = jnp.max(jnp.where(ohf > 0, m, _FMIN), axis=1, keepdims=True)
        old = segmax_s[...]
        new = jnp.maximum(old, lm)
        segmax_s[...] = new
        scale = jnp.exp(old - new)
        smg = jax.lax.dot_general(
            new.T, ohf, (((1,), (0,)), ((), ())),
            preferred_element_type=jnp.float32)
        w = jnp.exp(m - smg)
        wog = ohf * w
        denom_s[...] = denom_s[...] * scale + jnp.sum(wog, axis=1,
                                                      keepdims=True)
        fsum_s[...] = fsum_s[...] * scale + jax.lax.dot_general(
            wog, xb, (((1,), (1,)), ((), ())),
            preferred_element_type=jnp.float32)

    @pl.when(ph == 1)
    def _phase1():
        xb = xstash[nb]
        am = am_s[nb].reshape(1, -1)
        denom = denom_s[...]
        cls = fsum_s[...] / jnp.where(denom > 0, denom, 1.0)
        kio = jax.lax.broadcasted_iota(jnp.int32, (K,) + am.shape[1:], 0)
        ohf = (kio == am).astype(jnp.float32)
        sl = jax.lax.dot_general(
            cls, ohf, (((0,), (0,)), ((), ())),
            preferred_element_type=jnp.float32)
        out_ref[0] = xb * sl


def kernel(x, preds):
    B, C, H, W, D = x.shape
    K = preds.shape[1]
    N = H * W * D
    Nb = 16384
    NB = N // Nb
    xf = x.reshape(B, C, N)
    pf = preds.reshape(B, K, N)

    out = pl.pallas_call(
        functools.partial(_fused_body, K=K),
        grid=(B, 2, NB),
        in_specs=[
            pl.BlockSpec((1, K, Nb),
                         lambda b, ph, n: (b, 0, jnp.where(ph == 0, n, 0))),
            pl.BlockSpec((1, C, Nb),
                         lambda b, ph, n: (b, 0, jnp.where(ph == 0, n, 0))),
        ],
        out_specs=pl.BlockSpec(
            (1, C, Nb), lambda b, ph, n: (b, 0, jnp.where(ph == 1, n, 0))),
        out_shape=jax.ShapeDtypeStruct((B, C, N), jnp.float32),
        scratch_shapes=[
            pltpu.VMEM((NB, C, Nb), jnp.float32),
            pltpu.VMEM((NB, 8, Nb // 8), jnp.int32),
            pltpu.VMEM((K, 1), jnp.float32),
            pltpu.VMEM((K, 1), jnp.float32),
            pltpu.VMEM((K, C), jnp.float32),
        ],
    )(pf, xf)

    return out.reshape(B, C, H, W, D)


# speedup vs baseline: 1.2639x; 1.0422x over previous
import functools

import jax
import jax.numpy as jnp
from jax.experimental import pallas as pl
from jax.experimental.pallas import tpu as pltpu

_FMIN = float(jnp.finfo(jnp.float32).min)


def _fused_body(preds_ref, x_ref, out_ref, xstash, am_s, segmax_s,
                denom_s, fsum_s, *, K):
    ph = pl.program_id(1)
    nb = pl.program_id(2)

    @pl.when(ph == 0)
    def _phase0():
        @pl.when(nb == 0)
        def _():
            segmax_s[...] = jnp.full(segmax_s.shape, _FMIN, jnp.float32)
            denom_s[...] = jnp.zeros(denom_s.shape, jnp.float32)
            fsum_s[...] = jnp.zeros(fsum_s.shape, jnp.float32)

        p = preds_ref[0]
        xb = x_ref[0]
        xstash[nb] = xb
        m = jnp.max(p, axis=0, keepdims=True)
        kio = jax.lax.broadcasted_iota(jnp.int32, p.shape, 0)
        am = jnp.min(jnp.where(p == m, kio, K), axis=0, keepdims=True)
        am_s[nb] = am.reshape(am_s.shape[1:])
        ohf = (kio == am).astype(jnp.float32)
        lm = jnp.max(jnp.where(ohf > 0, m, _FMIN), axis=1, keepdims=True)
        old = segmax_s[...]
        new = jnp.maximum(old, lm)
        segmax_s[...] = new
        scale = jnp.exp(old - new)
        smg = jax.lax.dot_general(
            new.T, ohf, (((1,), (0,)), ((), ())),
            preferred_element_type=jnp.float32)
        w = jnp.exp(m - smg)
        wog = ohf * w
        denom_s[...] = denom_s[...] * scale + jnp.sum(wog, axis=1,
                                                      keepdims=True)
        fsum_s[...] = fsum_s[...] * scale + jax.lax.dot_general(
            wog, xb, (((1,), (1,)), ((), ())),
            preferred_element_type=jnp.float32)

    @pl.when(ph == 1)
    def _phase1():
        xb = xstash[nb]
        am = am_s[nb].reshape(1, -1)
        denom = denom_s[...]
        cls = fsum_s[...] / jnp.where(denom > 0, denom, 1.0)
        kio = jax.lax.broadcasted_iota(jnp.int32, (K,) + am.shape[1:], 0)
        ohf = (kio == am).astype(jnp.float32)
        sl = jax.lax.dot_general(
            cls, ohf, (((0,), (0,)), ((), ())),
            preferred_element_type=jnp.float32)
        out_ref[0] = xb * sl


def kernel(x, preds):
    B, C, H, W, D = x.shape
    K = preds.shape[1]
    N = H * W * D
    Nb = 32768
    NB = N // Nb
    xf = x.reshape(B, C, N)
    pf = preds.reshape(B, K, N)

    out = pl.pallas_call(
        functools.partial(_fused_body, K=K),
        grid=(B, 2, NB),
        in_specs=[
            pl.BlockSpec((1, K, Nb),
                         lambda b, ph, n: (b, 0, jnp.where(ph == 0, n, 0))),
            pl.BlockSpec((1, C, Nb),
                         lambda b, ph, n: (b, 0, jnp.where(ph == 0, n, 0))),
        ],
        out_specs=pl.BlockSpec(
            (1, C, Nb), lambda b, ph, n: (b, 0, jnp.where(ph == 1, n, 0))),
        out_shape=jax.ShapeDtypeStruct((B, C, N), jnp.float32),
        scratch_shapes=[
            pltpu.VMEM((NB, C, Nb), jnp.float32),
            pltpu.VMEM((NB, 8, Nb // 8), jnp.int32),
            pltpu.VMEM((K, 1), jnp.float32),
            pltpu.VMEM((K, 1), jnp.float32),
            pltpu.VMEM((K, C), jnp.float32),
        ],
    )(pf, xf)

    return out.reshape(B, C, H, W, D)
